# baseline (speedup 1.0000x reference)
import functools

import jax
import jax.numpy as jnp
from jax import lax
from jax.experimental import pallas as pl
from jax.experimental.pallas import tpu as pltpu
from jax.experimental.pallas import tpu_sc as plsc

_NC = 2
_NS = 16
_NW = _NC * _NS
_CHUNK = 32
_LANES = 16


def _sc_body(batch, seq_len, d_model, rows_per_w, x_hbm, tab_hbm, out_hbm,
             t_buf, x_buf):
    c = lax.axis_index("c")
    s = lax.axis_index("s")
    wid = s * _NC + c
    base = wid * rows_per_w
    celems = _CHUNK * d_model
    for ci in range(rows_per_w // _CHUNK):
        toff = (base + ci * _CHUNK) * d_model
        pltpu.sync_copy(tab_hbm.at[pl.ds(toff, celems)], t_buf)
        for b in range(batch):
            xoff = (b * seq_len + base + ci * _CHUNK) * d_model
            pltpu.sync_copy(x_hbm.at[pl.ds(xoff, celems)], x_buf)

            @plsc.parallel_loop(0, celems // _LANES, unroll=8)
            def _add(i):
                off = i * _LANES
                plsc.addupdate(x_buf.at[pl.ds(off, _LANES)],
                               t_buf[pl.ds(off, _LANES)])

            pltpu.sync_copy(x_buf, out_hbm.at[pl.ds(xoff, celems)])


def kernel(x, pos_table):
    batch, seq_len, d_model = x.shape
    rows_per_w = seq_len // _NW
    xf = x.reshape(batch * seq_len * d_model)
    tf = pos_table[:seq_len].reshape(seq_len * d_model)
    mesh = plsc.VectorSubcoreMesh(core_axis_name="c", subcore_axis_name="s")
    sc_add = pl.kernel(
        functools.partial(_sc_body, batch, seq_len, d_model, rows_per_w),
        out_type=jax.ShapeDtypeStruct((batch * seq_len * d_model,), x.dtype),
        mesh=mesh,
        scratch_types=[
            pltpu.VMEM((_CHUNK * d_model,), jnp.float32),
            pltpu.VMEM((_CHUNK * d_model,), jnp.float32),
        ],
    )
    out = sc_add(xf, tf)
    return out.reshape(batch, seq_len, d_model)


# speedup vs baseline: 1.0744x; 1.0744x over previous
import functools

import jax
import jax.numpy as jnp
from jax import lax
from jax.experimental import pallas as pl
from jax.experimental.pallas import tpu as pltpu
from jax.experimental.pallas import tpu_sc as plsc

_NC = 2
_NS = 16
_NW = _NC * _NS
_CHUNK = 32
_LANES = 16


def _sc_body(batch, seq_len, d_model, rows_per_w, x_hbm, tab_hbm, out_hbm,
             t_bufs, x_bufs, in_sem, out_sem, t_sem):
    c = lax.axis_index("c")
    s = lax.axis_index("s")
    wid = s * _NC + c
    base = wid * rows_per_w
    celems = _CHUNK * d_model
    n_ci = rows_per_w // _CHUNK
    n_steps = n_ci * batch

    def xoff(k):
        ci, b = divmod(k, batch)
        return (b * seq_len + base + ci * _CHUNK) * d_model

    def issue_t(ci):
        toff = (base + ci * _CHUNK) * d_model
        return pltpu.async_copy(tab_hbm.at[pl.ds(toff, celems)],
                                t_bufs.at[ci % 2], t_sem.at[ci % 2])

    def issue_in(k):
        return pltpu.async_copy(x_hbm.at[pl.ds(xoff(k), celems)],
                                x_bufs.at[k % 2], in_sem.at[k % 2])

    t_descs = [issue_t(0)]
    in_desc = [issue_in(0)]
    out_desc = []
    for k in range(n_steps):
        ci, b = divmod(k, batch)
        buf = k % 2
        if b == 0 and ci + 1 < n_ci:
            t_descs.append(issue_t(ci + 1))
        in_desc[k].wait()
        if k >= 1:
            out_desc[k - 1].wait()
        if k + 1 < n_steps:
            in_desc.append(issue_in(k + 1))
        if b == 0:
            t_descs[ci].wait()
        t_buf = t_bufs.at[ci % 2]
        x_buf = x_bufs.at[buf]

        @plsc.parallel_loop(0, celems // _LANES, unroll=8)
        def _add(i):
            off = i * _LANES
            plsc.addupdate(x_buf.at[pl.ds(off, _LANES)],
                           t_buf[pl.ds(off, _LANES)])

        out_desc.append(
            pltpu.async_copy(x_bufs.at[buf],
                             out_hbm.at[pl.ds(xoff(k), celems)],
                             out_sem.at[buf]))
    out_desc[n_steps - 1].wait()


def kernel(x, pos_table):
    batch, seq_len, d_model = x.shape
    rows_per_w = seq_len // _NW
    xf = x.reshape(batch * seq_len * d_model)
    tf = pos_table[:seq_len].reshape(seq_len * d_model)
    mesh = plsc.VectorSubcoreMesh(core_axis_name="c", subcore_axis_name="s")
    sc_add = pl.kernel(
        functools.partial(_sc_body, batch, seq_len, d_model, rows_per_w),
        out_type=jax.ShapeDtypeStruct((batch * seq_len * d_model,), x.dtype),
        mesh=mesh,
        scratch_types=[
            pltpu.VMEM((2, _CHUNK * d_model), jnp.float32),
            pltpu.VMEM((2, _CHUNK * d_model), jnp.float32),
            pltpu.SemaphoreType.DMA((2,)),
            pltpu.SemaphoreType.DMA((2,)),
            pltpu.SemaphoreType.DMA((2,)),
        ],
    )
    out = sc_add(xf, tf)
    return out.reshape(batch, seq_len, d_model)


# speedup vs baseline: 3.3619x; 3.1293x over previous
import functools

import jax
import jax.numpy as jnp
from jax import lax
from jax.experimental import pallas as pl
from jax.experimental.pallas import tpu as pltpu
from jax.experimental.pallas import tpu_sc as plsc

_NC = 2
_NS = 16
_NW = _NC * _NS
_CHUNK = 32
_LANES = 16


def _sc_body(batch, seq_len, d_model, rows_per_w, x_hbm, tab_hbm, out_hbm,
             t_bufs, x_bufs, in_sem, out_sem, t_sem):
    c = lax.axis_index("c")
    s = lax.axis_index("s")
    wid = s * _NC + c
    base = wid * rows_per_w
    n_ci = rows_per_w // _CHUNK
    n_steps = n_ci * batch

    def xrow(k):
        ci, b = divmod(k, batch)
        return b * seq_len + base + ci * _CHUNK

    def issue_t(ci):
        row = base + ci * _CHUNK
        return pltpu.async_copy(tab_hbm.at[pl.ds(row, _CHUNK), :],
                                t_bufs.at[ci % 2], t_sem.at[ci % 2])

    def issue_in(k):
        return pltpu.async_copy(x_hbm.at[pl.ds(xrow(k), _CHUNK), :],
                                x_bufs.at[k % 2], in_sem.at[k % 2])

    t_descs = [issue_t(0)]
    in_desc = [issue_in(0)]
    out_desc = []
    for k in range(n_steps):
        ci, b = divmod(k, batch)
        buf = k % 2
        if b == 0 and ci + 1 < n_ci:
            t_descs.append(issue_t(ci + 1))
        in_desc[k].wait()
        if k >= 1:
            out_desc[k - 1].wait()
        if k + 1 < n_steps:
            in_desc.append(issue_in(k + 1))
        if b == 0:
            t_descs[ci].wait()
        t_buf = t_bufs.at[ci % 2]
        x_buf = x_bufs.at[buf]
        vecs_per_row = d_model // _LANES

        @plsc.parallel_loop(0, _CHUNK * vecs_per_row, unroll=8)
        def _add(i):
            r = i // vecs_per_row
            off = (i % vecs_per_row) * _LANES
            plsc.addupdate(x_buf.at[r, pl.ds(off, _LANES)],
                           t_buf[r, pl.ds(off, _LANES)])

        out_desc.append(
            pltpu.async_copy(x_bufs.at[buf],
                             out_hbm.at[pl.ds(xrow(k), _CHUNK), :],
                             out_sem.at[buf]))
    out_desc[n_steps - 1].wait()


def kernel(x, pos_table):
    batch, seq_len, d_model = x.shape
    rows_per_w = seq_len // _NW
    x2 = x.reshape(batch * seq_len, d_model)
    mesh = plsc.VectorSubcoreMesh(core_axis_name="c", subcore_axis_name="s")
    sc_add = pl.kernel(
        functools.partial(_sc_body, batch, seq_len, d_model, rows_per_w),
        out_type=jax.ShapeDtypeStruct((batch * seq_len, d_model), x.dtype),
        mesh=mesh,
        scratch_types=[
            pltpu.VMEM((2, _CHUNK, d_model), jnp.float32),
            pltpu.VMEM((2, _CHUNK, d_model), jnp.float32),
            pltpu.SemaphoreType.DMA((2,)),
            pltpu.SemaphoreType.DMA((2,)),
            pltpu.SemaphoreType.DMA((2,)),
        ],
    )
    out = sc_add(x2, pos_table[:seq_len])
    return out.reshape(batch, seq_len, d_model)
